# baseline (speedup 1.0000x reference)
import functools

import jax
import jax.numpy as jnp
from jax import lax
from jax.experimental import pallas as pl
from jax.experimental.pallas import tpu as pltpu
from jax.experimental.pallas import tpu_sc as plsc

N = 10000
E = 320000
F = 128
NCLASS = 40

NC = 2
NS = 16
W = 125
EPT = E // (NC * NS)
WPT = EPT // W
SGRP = 2
NGRP = WPT // SGRP
SUPER = NGRP // 2
NPAD = 10240
RPT = NPAD // NS
ZR = 8

_mesh = plsc.VectorSubcoreMesh(core_axis_name="c", subcore_axis_name="s")


def _zero_vmem_2d(ref, nrows, ncols):
    z16 = jnp.zeros((16,), jnp.float32)

    @pl.loop(0, nrows)
    def _(r):
        @pl.loop(0, ncols, step=16)
        def _(col):
            ref[r, pl.ds(col, 16)] = z16


def _fill_ones(ref, n):
    o16 = jnp.ones((16,), jnp.float32)

    @pl.loop(0, n - 15, step=16)
    def _(i):
        ref[pl.ds(i, 16)] = o16

    ref[pl.ds(n - 16, 16)] = o16


def _zero_vmem_1d(ref, n):
    z16 = jnp.zeros((16,), jnp.float32)

    @pl.loop(0, n, step=16)
    def _(i):
        ref[pl.ds(i, 16)] = z16


@jax.jit
def _sc_degrees(srcg, dstg):
    @functools.partial(
        pl.kernel,
        out_type=(
            jax.ShapeDtypeStruct((NC * NPAD,), jnp.float32),
            jax.ShapeDtypeStruct((NC * NPAD,), jnp.float32),
        ),
        mesh=_mesh,
        scratch_types=(
            [pltpu.VMEM((SGRP, W), jnp.int32)] * 4
            + [
                pltpu.VMEM((128,), jnp.float32),
                pltpu.VMEM((RPT,), jnp.float32),
                pltpu.VMEM_SHARED((NPAD,), jnp.float32),
                pltpu.VMEM_SHARED((NPAD,), jnp.float32),
            ]
            + [pltpu.SemaphoreType.DMA] * 5
        ),
    )
    def k(srcg_hbm, dstg_hbm, do_hbm, di_hbm, sb0, sb1, db0, db1, ones_v, zb,
          do_sh, di_sh, is0, is1, id0, id1, dsem):
        sb = (sb0, sb1)
        db = (db0, db1)
        isem_s = (is0, is1)
        isem_d = (id0, id1)
        cid = lax.axis_index("c")
        sid = lax.axis_index("s")
        wid = cid * NS + sid
        rowbase = wid * NGRP

        _zero_vmem_1d(zb, RPT)
        pltpu.sync_copy(zb, do_sh.at[pl.ds(sid * RPT, RPT)])
        pltpu.sync_copy(zb, di_sh.at[pl.ds(sid * RPT, RPT)])

        _fill_ones(ones_v, 128)
        ones_w = ones_v.at[pl.ds(0, W)]

        pltpu.sync_copy(srcg_hbm.at[rowbase], sb[0])
        pltpu.sync_copy(dstg_hbm.at[rowbase], db[0])
        pltpu.sync_copy(srcg_hbm.at[rowbase + 1], sb[1])
        pltpu.sync_copy(dstg_hbm.at[rowbase + 1], db[1])

        plsc.subcore_barrier()

        @pl.loop(0, SUPER)
        def _(s):
            for phase in range(2):
                r = s * 2 + phase

                @pl.when(r >= 2)
                def _():
                    pltpu.make_async_copy(srcg_hbm.at[rowbase], sb[phase],
                                          isem_s[phase]).wait()
                    pltpu.make_async_copy(dstg_hbm.at[rowbase], db[phase],
                                          isem_d[phase]).wait()

                for j in range(SGRP):
                    pltpu.async_copy(ones_w, do_sh.at[sb[phase].at[j]], dsem,
                                     add=True)
                    pltpu.async_copy(ones_w, di_sh.at[db[phase].at[j]], dsem,
                                     add=True)
                for _j in range(2 * SGRP):
                    pltpu.make_async_copy(ones_w, do_sh.at[sb[0].at[0]],
                                          dsem).wait()

                @pl.when(r + 2 < NGRP)
                def _():
                    pltpu.async_copy(srcg_hbm.at[rowbase + r + 2], sb[phase],
                                     isem_s[phase])
                    pltpu.async_copy(dstg_hbm.at[rowbase + r + 2], db[phase],
                                     isem_d[phase])

        plsc.subcore_barrier()

        base = cid * NPAD + sid * RPT
        pltpu.sync_copy(do_sh.at[pl.ds(sid * RPT, RPT)], do_hbm.at[pl.ds(base, RPT)])
        pltpu.sync_copy(di_sh.at[pl.ds(sid * RPT, RPT)], di_hbm.at[pl.ds(base, RPT)])

    return k(srcg, dstg)


@jax.jit
def _sc_spmm(srcg, dstg, hs):
    @functools.partial(
        pl.kernel,
        out_type=jax.ShapeDtypeStruct((NC * NPAD, F), jnp.float32),
        mesh=_mesh,
        scratch_types=(
            [pltpu.VMEM((SGRP, W), jnp.int32)] * 4
            + [pltpu.VMEM((W, F), jnp.float32)] * SGRP
            + [
                pltpu.VMEM((ZR, F), jnp.float32),
                pltpu.VMEM_SHARED((NPAD, F), jnp.float32),
            ]
            + [pltpu.SemaphoreType.DMA] * (4 + 2 * SGRP)
        ),
    )
    def k(srcg_hbm, dstg_hbm, hs_hbm, out_hbm, *rest):
        sb = rest[0:2]
        db = rest[2:4]
        rows = rest[4:4 + SGRP]
        zb = rest[4 + SGRP]
        agg_sh = rest[5 + SGRP]
        sems = rest[6 + SGRP:]
        isem_s = sems[0:2]
        isem_d = sems[2:4]
        gsem = sems[4:4 + SGRP]
        ssem = sems[4 + SGRP:]

        cid = lax.axis_index("c")
        sid = lax.axis_index("s")
        wid = cid * NS + sid
        rowbase = wid * NGRP

        _zero_vmem_2d(zb, ZR, F)

        @pl.loop(0, RPT, step=ZR)
        def _(r):
            pltpu.sync_copy(zb, agg_sh.at[pl.ds(sid * RPT + r, ZR)])

        pltpu.sync_copy(srcg_hbm.at[rowbase], sb[0])
        pltpu.sync_copy(dstg_hbm.at[rowbase], db[0])
        pltpu.sync_copy(srcg_hbm.at[rowbase + 1], sb[1])
        pltpu.sync_copy(dstg_hbm.at[rowbase + 1], db[1])

        def gather(slot, idx_ref):
            pltpu.async_copy(hs_hbm.at[idx_ref], rows[slot], gsem[slot])

        def gather_wait(slot):
            pltpu.make_async_copy(hs_hbm.at[sb[0].at[0]], rows[slot],
                                  gsem[slot]).wait()

        def scatter(slot, idx_ref):
            pltpu.async_copy(rows[slot], agg_sh.at[idx_ref], ssem[slot],
                             add=True)

        def scatter_wait(slot):
            pltpu.make_async_copy(rows[slot], agg_sh.at[db[0].at[0]],
                                  ssem[slot]).wait()

        def idx_prefetch(phase, grp):
            pltpu.async_copy(srcg_hbm.at[rowbase + grp], sb[phase],
                             isem_s[phase])
            pltpu.async_copy(dstg_hbm.at[rowbase + grp], db[phase],
                             isem_d[phase])

        def idx_wait_src(phase):
            pltpu.make_async_copy(srcg_hbm.at[rowbase], sb[phase],
                                  isem_s[phase]).wait()

        def idx_wait_dst(phase):
            pltpu.make_async_copy(dstg_hbm.at[rowbase], db[phase],
                                  isem_d[phase]).wait()

        plsc.subcore_barrier()

        for j in range(SGRP):
            gather(j, sb[0].at[j])

        @pl.loop(0, SUPER)
        def _(s):
            for phase in range(2):
                r = s * 2 + phase
                @pl.when(r >= 2)
                def _():
                    idx_wait_dst(phase)

                for j in range(SGRP):
                    gather_wait(j)
                    scatter(j, db[phase].at[j])
                @pl.when(jnp.logical_and(r >= 1, r + 1 < NGRP))
                def _():
                    idx_wait_src(1 - phase)

                for j in range(SGRP):
                    scatter_wait(j)

                    @pl.when(r + 1 < NGRP)
                    def _():
                        gather(j, sb[1 - phase].at[j])

                @pl.when(r + 2 < NGRP)
                def _():
                    idx_prefetch(phase, r + 2)

        plsc.subcore_barrier()

        base = cid * NPAD + sid * RPT
        pltpu.sync_copy(agg_sh.at[pl.ds(sid * RPT, RPT)],
                        out_hbm.at[pl.ds(base, RPT)])

    return k(srcg, dstg, hs)


def _norm(deg2):
    d = deg2[0] + deg2[1]
    return jnp.where(d > 0.0, lax.rsqrt(jnp.maximum(d, 1.0)), 0.0)


def _tc_matmul_scale_body(x_ref, w_ref, dego_ref, o_ref):
    ns = _norm(dego_ref[...])
    h = jnp.dot(x_ref[...], w_ref[...], preferred_element_type=jnp.float32)
    o_ref[...] = h * ns


def _tc_mid_body(p_ref, degi_ref, dego_ref, b0_ref, w1_ref, o_ref):
    agg = p_ref[0] + p_ref[1]
    nd = _norm(degi_ref[...])
    t = jax.nn.relu(agg * nd + b0_ref[...])
    ns = _norm(dego_ref[...])
    o_ref[...] = jnp.dot(t, w1_ref[...], preferred_element_type=jnp.float32) * ns


def _tc_final_body(q_ref, degi_ref, b1_ref, wo_ref, bo_ref, o_ref):
    agg = q_ref[0] + q_ref[1]
    nd = _norm(degi_ref[...])
    u = jax.nn.relu(agg * nd + b1_ref[...])
    logits = jnp.dot(u, wo_ref[...], preferred_element_type=jnp.float32) + bo_ref[...]
    m = jnp.max(logits, axis=1, keepdims=True)
    e = jnp.exp(logits - m)
    lse = jnp.log(jnp.sum(e, axis=1, keepdims=True)) + m
    o_ref[...] = logits - lse


_BR = 1000
_NB = N // _BR


def _row_specs():
    return {
        "x": pl.BlockSpec((_BR, F), lambda i: (i, 0)),
        "p": pl.BlockSpec((2, _BR, F), lambda i: (0, i, 0)),
        "deg": pl.BlockSpec((2, _BR, 1), lambda i: (0, i, 0)),
        "w": pl.BlockSpec((F, F), lambda i: (0, 0)),
        "b": pl.BlockSpec((1, F), lambda i: (0, 0)),
        "o": pl.BlockSpec((_BR, F), lambda i: (i, 0)),
    }


@jax.jit
def _tc_matmul_scale(x, W0, dego):
    s = _row_specs()
    return pl.pallas_call(
        _tc_matmul_scale_body,
        grid=(_NB,),
        in_specs=[s["x"], s["w"], s["deg"]],
        out_specs=s["o"],
        out_shape=jax.ShapeDtypeStruct((N, F), jnp.float32),
    )(x, W0, dego)


@jax.jit
def _tc_mid(p, degi, dego, b0, W1):
    s = _row_specs()
    return pl.pallas_call(
        _tc_mid_body,
        grid=(_NB,),
        in_specs=[s["p"], s["deg"], s["deg"], s["b"], s["w"]],
        out_specs=s["o"],
        out_shape=jax.ShapeDtypeStruct((N, F), jnp.float32),
    )(p, degi, dego, b0, W1)


@jax.jit
def _tc_final(q, degi, b1, wo, bo):
    s = _row_specs()
    return pl.pallas_call(
        _tc_final_body,
        grid=(_NB,),
        in_specs=[s["p"], s["deg"], s["b"], s["w"], s["b"]],
        out_specs=s["o"],
        out_shape=jax.ShapeDtypeStruct((N, F), jnp.float32),
    )(q, degi, b1, wo, bo)


def kernel(graph, x, W0, b0, W1, b1, W_out, b_out):
    srcg = graph[0].reshape(NC * NS * NGRP, SGRP, W)
    dstg = graph[1].reshape(NC * NS * NGRP, SGRP, W)

    dego_p, degi_p = _sc_degrees(srcg, dstg)
    dego = dego_p.reshape(NC, NPAD, 1)
    degi = degi_p.reshape(NC, NPAD, 1)

    h0 = _tc_matmul_scale(x, W0, dego)
    p = _sc_spmm(srcg, dstg, h0).reshape(NC, NPAD, F)
    h1 = _tc_mid(p, degi, dego, b0.reshape(1, F), W1)
    q = _sc_spmm(srcg, dstg, h1).reshape(NC, NPAD, F)

    wo = jnp.zeros((F, F), jnp.float32).at[:, :NCLASS].set(W_out)
    bo = jnp.full((F,), -1e9, jnp.float32).at[:NCLASS].set(b_out).reshape(1, F)
    out = _tc_final(q, degi, b1.reshape(1, F), wo, bo)
    return out[:, :NCLASS]


# speedup vs baseline: 1.0212x; 1.0212x over previous
import functools

import jax
import jax.numpy as jnp
from jax import lax
from jax.experimental import pallas as pl
from jax.experimental.pallas import tpu as pltpu
from jax.experimental.pallas import tpu_sc as plsc

N = 10000
E = 320000
F = 128
NCLASS = 40

NC = 2
NS = 16
W = 125
EPT = E // (NC * NS)
WPT = EPT // W
SGRP = 2
NGRP = WPT // SGRP
SUPER = NGRP // 2
NPAD = 10240
RPT = NPAD // NS
ZR = 16

_mesh = plsc.VectorSubcoreMesh(core_axis_name="c", subcore_axis_name="s")


def _zero_vmem_2d(ref, nrows, ncols):
    z16 = jnp.zeros((16,), jnp.float32)

    @pl.loop(0, nrows)
    def _(r):
        @pl.loop(0, ncols, step=16)
        def _(col):
            ref[r, pl.ds(col, 16)] = z16


def _fill_ones(ref, n):
    o16 = jnp.ones((16,), jnp.float32)

    @pl.loop(0, n - 15, step=16)
    def _(i):
        ref[pl.ds(i, 16)] = o16

    ref[pl.ds(n - 16, 16)] = o16


def _zero_vmem_1d(ref, n):
    z16 = jnp.zeros((16,), jnp.float32)

    @pl.loop(0, n, step=16)
    def _(i):
        ref[pl.ds(i, 16)] = z16


@jax.jit
def _sc_degrees(srcg, dstg):
    @functools.partial(
        pl.kernel,
        out_type=(
            jax.ShapeDtypeStruct((NC * NPAD,), jnp.float32),
            jax.ShapeDtypeStruct((NC * NPAD,), jnp.float32),
        ),
        mesh=_mesh,
        scratch_types=(
            [pltpu.VMEM((SGRP, W), jnp.int32)] * 4
            + [
                pltpu.VMEM((128,), jnp.float32),
                pltpu.VMEM((RPT,), jnp.float32),
                pltpu.VMEM_SHARED((NPAD,), jnp.float32),
                pltpu.VMEM_SHARED((NPAD,), jnp.float32),
            ]
            + [pltpu.SemaphoreType.DMA] * 5
        ),
    )
    def k(srcg_hbm, dstg_hbm, do_hbm, di_hbm, sb0, sb1, db0, db1, ones_v, zb,
          do_sh, di_sh, is0, is1, id0, id1, dsem):
        sb = (sb0, sb1)
        db = (db0, db1)
        isem_s = (is0, is1)
        isem_d = (id0, id1)
        cid = lax.axis_index("c")
        sid = lax.axis_index("s")
        wid = cid * NS + sid
        rowbase = wid * NGRP

        _zero_vmem_1d(zb, RPT)
        pltpu.sync_copy(zb, do_sh.at[pl.ds(sid * RPT, RPT)])
        pltpu.sync_copy(zb, di_sh.at[pl.ds(sid * RPT, RPT)])

        _fill_ones(ones_v, 128)
        ones_w = ones_v.at[pl.ds(0, W)]

        pltpu.sync_copy(srcg_hbm.at[rowbase], sb[0])
        pltpu.sync_copy(dstg_hbm.at[rowbase], db[0])
        pltpu.sync_copy(srcg_hbm.at[rowbase + 1], sb[1])
        pltpu.sync_copy(dstg_hbm.at[rowbase + 1], db[1])

        plsc.subcore_barrier()

        @pl.loop(0, SUPER)
        def _(s):
            for phase in range(2):
                r = s * 2 + phase

                @pl.when(r >= 2)
                def _():
                    pltpu.make_async_copy(srcg_hbm.at[rowbase], sb[phase],
                                          isem_s[phase]).wait()
                    pltpu.make_async_copy(dstg_hbm.at[rowbase], db[phase],
                                          isem_d[phase]).wait()

                for j in range(SGRP):
                    pltpu.async_copy(ones_w, do_sh.at[sb[phase].at[j]], dsem,
                                     add=True)
                    pltpu.async_copy(ones_w, di_sh.at[db[phase].at[j]], dsem,
                                     add=True)
                for _j in range(2 * SGRP):
                    pltpu.make_async_copy(ones_w, do_sh.at[sb[0].at[0]],
                                          dsem).wait()

                @pl.when(r + 2 < NGRP)
                def _():
                    pltpu.async_copy(srcg_hbm.at[rowbase + r + 2], sb[phase],
                                     isem_s[phase])
                    pltpu.async_copy(dstg_hbm.at[rowbase + r + 2], db[phase],
                                     isem_d[phase])

        plsc.subcore_barrier()

        base = cid * NPAD + sid * RPT
        pltpu.sync_copy(do_sh.at[pl.ds(sid * RPT, RPT)], do_hbm.at[pl.ds(base, RPT)])
        pltpu.sync_copy(di_sh.at[pl.ds(sid * RPT, RPT)], di_hbm.at[pl.ds(base, RPT)])

    return k(srcg, dstg)


@jax.jit
def _sc_spmm(srcg, dstg, hs):
    @functools.partial(
        pl.kernel,
        out_type=jax.ShapeDtypeStruct((NC * NPAD, F), jnp.float32),
        mesh=_mesh,
        scratch_types=(
            [pltpu.VMEM((SGRP, W), jnp.int32)] * 4
            + [pltpu.VMEM((W, F), jnp.float32)] * SGRP
            + [
                pltpu.VMEM((ZR, F), jnp.float32),
                pltpu.VMEM_SHARED((NPAD, F), jnp.float32),
            ]
            + [pltpu.SemaphoreType.DMA] * (5 + 2 * SGRP)
        ),
    )
    def k(srcg_hbm, dstg_hbm, hs_hbm, out_hbm, *rest):
        sb = rest[0:2]
        db = rest[2:4]
        rows = rest[4:4 + SGRP]
        zb = rest[4 + SGRP]
        agg_sh = rest[5 + SGRP]
        sems = rest[6 + SGRP:]
        isem_s = sems[0:2]
        isem_d = sems[2:4]
        zsem = sems[4]
        gsem = sems[5:5 + SGRP]
        ssem = sems[5 + SGRP:]

        cid = lax.axis_index("c")
        sid = lax.axis_index("s")
        wid = cid * NS + sid
        rowbase = wid * NGRP

        _zero_vmem_2d(zb, ZR, F)

        @pl.loop(0, 2)
        def _(b):
            for kk in range(20):
                pltpu.async_copy(
                    zb, agg_sh.at[pl.ds(sid * RPT + (b * 20 + kk) * ZR, ZR)],
                    zsem)
            for _kk in range(20):
                pltpu.make_async_copy(zb, agg_sh.at[pl.ds(sid * RPT, ZR)],
                                      zsem).wait()

        pltpu.sync_copy(srcg_hbm.at[rowbase], sb[0])
        pltpu.sync_copy(dstg_hbm.at[rowbase], db[0])
        pltpu.sync_copy(srcg_hbm.at[rowbase + 1], sb[1])
        pltpu.sync_copy(dstg_hbm.at[rowbase + 1], db[1])

        def gather(slot, idx_ref):
            pltpu.async_copy(hs_hbm.at[idx_ref], rows[slot], gsem[slot])

        def gather_wait(slot):
            pltpu.make_async_copy(hs_hbm.at[sb[0].at[0]], rows[slot],
                                  gsem[slot]).wait()

        def scatter(slot, idx_ref):
            pltpu.async_copy(rows[slot], agg_sh.at[idx_ref], ssem[slot],
                             add=True)

        def scatter_wait(slot):
            pltpu.make_async_copy(rows[slot], agg_sh.at[db[0].at[0]],
                                  ssem[slot]).wait()

        def idx_prefetch(phase, grp):
            pltpu.async_copy(srcg_hbm.at[rowbase + grp], sb[phase],
                             isem_s[phase])
            pltpu.async_copy(dstg_hbm.at[rowbase + grp], db[phase],
                             isem_d[phase])

        def idx_wait_src(phase):
            pltpu.make_async_copy(srcg_hbm.at[rowbase], sb[phase],
                                  isem_s[phase]).wait()

        def idx_wait_dst(phase):
            pltpu.make_async_copy(dstg_hbm.at[rowbase], db[phase],
                                  isem_d[phase]).wait()

        plsc.subcore_barrier()

        for j in range(SGRP):
            gather(j, sb[0].at[j])

        @pl.loop(0, SUPER)
        def _(s):
            for phase in range(2):
                r = s * 2 + phase
                @pl.when(r >= 2)
                def _():
                    idx_wait_dst(phase)

                for j in range(SGRP):
                    gather_wait(j)
                    scatter(j, db[phase].at[j])
                @pl.when(jnp.logical_and(r >= 1, r + 1 < NGRP))
                def _():
                    idx_wait_src(1 - phase)

                for j in range(SGRP):
                    scatter_wait(j)

                    @pl.when(r + 1 < NGRP)
                    def _():
                        gather(j, sb[1 - phase].at[j])

                @pl.when(r + 2 < NGRP)
                def _():
                    idx_prefetch(phase, r + 2)

        plsc.subcore_barrier()

        base = cid * NPAD + sid * RPT
        pltpu.sync_copy(agg_sh.at[pl.ds(sid * RPT, RPT)],
                        out_hbm.at[pl.ds(base, RPT)])

    return k(srcg, dstg, hs)


def _norm(deg2):
    d = deg2[0] + deg2[1]
    return jnp.where(d > 0.0, lax.rsqrt(jnp.maximum(d, 1.0)), 0.0)


def _tc_matmul_scale_body(x_ref, w_ref, dego_ref, o_ref):
    ns = _norm(dego_ref[...])
    h = jnp.dot(x_ref[...], w_ref[...], preferred_element_type=jnp.float32)
    o_ref[...] = h * ns


def _tc_mid_body(p_ref, degi_ref, dego_ref, b0_ref, w1_ref, o_ref):
    agg = p_ref[0] + p_ref[1]
    nd = _norm(degi_ref[...])
    t = jax.nn.relu(agg * nd + b0_ref[...])
    ns = _norm(dego_ref[...])
    o_ref[...] = jnp.dot(t, w1_ref[...], preferred_element_type=jnp.float32) * ns


def _tc_final_body(q_ref, degi_ref, b1_ref, wo_ref, bo_ref, o_ref):
    agg = q_ref[0] + q_ref[1]
    nd = _norm(degi_ref[...])
    u = jax.nn.relu(agg * nd + b1_ref[...])
    logits = jnp.dot(u, wo_ref[...], preferred_element_type=jnp.float32) + bo_ref[...]
    m = jnp.max(logits, axis=1, keepdims=True)
    e = jnp.exp(logits - m)
    lse = jnp.log(jnp.sum(e, axis=1, keepdims=True)) + m
    o_ref[...] = logits - lse


_BR = 1000
_NB = N // _BR


def _row_specs():
    return {
        "x": pl.BlockSpec((_BR, F), lambda i: (i, 0)),
        "p": pl.BlockSpec((2, _BR, F), lambda i: (0, i, 0)),
        "deg": pl.BlockSpec((2, _BR, 1), lambda i: (0, i, 0)),
        "w": pl.BlockSpec((F, F), lambda i: (0, 0)),
        "b": pl.BlockSpec((1, F), lambda i: (0, 0)),
        "o": pl.BlockSpec((_BR, F), lambda i: (i, 0)),
    }


@jax.jit
def _tc_matmul_scale(x, W0, dego):
    s = _row_specs()
    return pl.pallas_call(
        _tc_matmul_scale_body,
        grid=(_NB,),
        in_specs=[s["x"], s["w"], s["deg"]],
        out_specs=s["o"],
        out_shape=jax.ShapeDtypeStruct((N, F), jnp.float32),
    )(x, W0, dego)


@jax.jit
def _tc_mid(p, degi, dego, b0, W1):
    s = _row_specs()
    return pl.pallas_call(
        _tc_mid_body,
        grid=(_NB,),
        in_specs=[s["p"], s["deg"], s["deg"], s["b"], s["w"]],
        out_specs=s["o"],
        out_shape=jax.ShapeDtypeStruct((N, F), jnp.float32),
    )(p, degi, dego, b0, W1)


@jax.jit
def _tc_final(q, degi, b1, wo, bo):
    s = _row_specs()
    return pl.pallas_call(
        _tc_final_body,
        grid=(_NB,),
        in_specs=[s["p"], s["deg"], s["b"], s["w"], s["b"]],
        out_specs=s["o"],
        out_shape=jax.ShapeDtypeStruct((N, F), jnp.float32),
    )(q, degi, b1, wo, bo)


def kernel(graph, x, W0, b0, W1, b1, W_out, b_out):
    srcg = graph[0].reshape(NC * NS * NGRP, SGRP, W)
    dstg = graph[1].reshape(NC * NS * NGRP, SGRP, W)

    dego_p, degi_p = _sc_degrees(srcg, dstg)
    dego = dego_p.reshape(NC, NPAD, 1)
    degi = degi_p.reshape(NC, NPAD, 1)

    h0 = _tc_matmul_scale(x, W0, dego)
    p = _sc_spmm(srcg, dstg, h0).reshape(NC, NPAD, F)
    h1 = _tc_mid(p, degi, dego, b0.reshape(1, F), W1)
    q = _sc_spmm(srcg, dstg, h1).reshape(NC, NPAD, F)

    wo = jnp.zeros((F, F), jnp.float32).at[:, :NCLASS].set(W_out)
    bo = jnp.full((F,), -1e9, jnp.float32).at[:NCLASS].set(b_out).reshape(1, F)
    out = _tc_final(q, degi, b1.reshape(1, F), wo, bo)
    return out[:, :NCLASS]
